# baseline (speedup 1.0000x reference)
import functools

import jax
import jax.numpy as jnp
from jax import lax
from jax.experimental import pallas as pl
from jax.experimental.pallas import tpu as pltpu
from jax.experimental.pallas import tpu_sc as plsc

STRENGTH = 0.1

NC = 2
NS = 16
LANES = 16

NBUF = 4

EDGE_WIN = 80


def _mask_project_body(f0_ref, m0_ref, t0_ref, w0_ref, f1_ref, m1_ref, t1_ref,
                       w1_ref, mk0_ref, pj0_ref, mk1_ref, pj1_ref, *, block_rows):
    b = pl.program_id(0)

    def one(feat_ref, midx_ref, tok_ref, w_ref, masked_ref, proj_ref):
        feat = feat_ref[...]
        midx = midx_ref[0, :]
        rows = b * block_rows + lax.broadcasted_iota(
            jnp.int32, (block_rows, midx.shape[0]), 0)
        is_masked = jnp.any(rows == midx[None, :], axis=1)
        tok = tok_ref[0, :]
        masked = jnp.where(is_masked[:, None], tok[None, :], feat)
        masked_ref[...] = masked
        proj_ref[...] = lax.dot_general(
            masked, w_ref[...], (((1,), (1,)), ((), ())),
            preferred_element_type=jnp.float32,
            precision=lax.Precision.HIGHEST)

    one(f0_ref, m0_ref, t0_ref, w0_ref, mk0_ref, pj0_ref)
    one(f1_ref, m1_ref, t1_ref, w1_ref, mk1_ref, pj1_ref)


def _mask_project2(feat0, midx0, tok0, w0, feat1, midx1, tok1, w1,
                   block_rows=1000):
    n, d = feat0.shape
    nb = n // block_rows
    nmask = midx0.shape[0]
    fspec = pl.BlockSpec((block_rows, d), lambda b: (b, 0))
    mspec = pl.BlockSpec((1, nmask), lambda b: (0, 0))
    tspec = pl.BlockSpec((1, d), lambda b: (0, 0))
    wspec = pl.BlockSpec((d, d), lambda b: (0, 0))
    return pl.pallas_call(
        functools.partial(_mask_project_body, block_rows=block_rows),
        grid=(nb,),
        in_specs=[fspec, mspec, tspec, wspec, fspec, mspec, tspec, wspec],
        out_specs=[fspec, fspec, fspec, fspec],
        out_shape=[jax.ShapeDtypeStruct((n, d), jnp.float32)] * 4,
    )(feat0, midx0.reshape(1, nmask), tok0, w0,
      feat1, midx1.reshape(1, nmask), tok1, w1)


def _pack_edges_body(row_ref, col_ref, out_ref, *, n_win):
    r = row_ref[...]
    c = col_ref[...]
    br = r.shape[0]
    st = jnp.stack([r.reshape(br, n_win, EDGE_WIN),
                    c.reshape(br, n_win, EDGE_WIN)], axis=2)
    out_ref[...] = st.reshape(br, n_win * 2 * EDGE_WIN)


def _pack_edges(row, col, block_rows=8):
    e = row.shape[0]
    nw = NC * NS
    e_per_w = e // nw
    n_win = e_per_w // EDGE_WIN
    espec = pl.BlockSpec((block_rows, e_per_w), lambda b: (b, 0))
    out = pl.pallas_call(
        functools.partial(_pack_edges_body, n_win=n_win),
        grid=(nw // block_rows,),
        in_specs=[espec, espec],
        out_specs=pl.BlockSpec((block_rows, 2 * e_per_w), lambda b: (b, 0)),
        out_shape=jax.ShapeDtypeStruct((nw, 2 * e_per_w), jnp.int32),
    )(row.reshape(nw, e_per_w), col.reshape(nw, e_per_w))
    return out.reshape(2 * e)


def _spmm_body(proj0, proj1, edges0, val0, edges1, val1,
               part0, part1,
               *refs, n_pad, d, e):
    ebuf = refs[0:NBUF]
    vbuf = refs[NBUF:2 * NBUF]
    sidx = refs[2 * NBUF:3 * NBUF]
    rows = refs[3 * NBUF:4 * NBUF]
    zbuf = refs[4 * NBUF]
    acc = refs[4 * NBUF + 1]
    semi = refs[4 * NBUF + 2:4 * NBUF + 2 + NBUF]
    semg = refs[4 * NBUF + 2 + NBUF:4 * NBUF + 2 + 2 * NBUF]
    sems = refs[4 * NBUF + 2 + 2 * NBUF:4 * NBUF + 2 + 3 * NBUF]

    c = lax.axis_index("c")
    s = lax.axis_index("s")
    wid = c * NS + s
    e_per_w = e // (NC * NS)
    n_win = e_per_w // EDGE_WIN
    rows_per_sub = n_pad // NS
    zrows = zbuf.shape[0]
    n_grp = EDGE_WIN // LANES
    n_peel = NBUF + 1

    def zfill(r, _):
        for j in range(d // LANES):
            zbuf[r, pl.ds(j * LANES, LANES)] = jnp.zeros((LANES,), jnp.float32)
        return 0
    lax.fori_loop(0, zrows, zfill, 0)

    def run_path(proj, edges, val, part):
        base0 = wid * n_win * 2 * EDGE_WIN
        vbase0 = wid * e_per_w

        def fire_idx(w, b):
            off = base0 + w * 2 * EDGE_WIN
            pltpu.async_copy(edges.at[pl.ds(off, 2 * EDGE_WIN)], ebuf[b], semi[b])
            pltpu.async_copy(val.at[pl.ds(vbase0 + w * EDGE_WIN, EDGE_WIN)],
                             vbuf[b], semi[b])

        def wait_idx(w, b):
            off = base0 + w * 2 * EDGE_WIN
            pltpu.make_async_copy(edges.at[pl.ds(off, 2 * EDGE_WIN)], ebuf[b],
                                  semi[b]).wait()
            pltpu.make_async_copy(val.at[pl.ds(vbase0 + w * EDGE_WIN, EDGE_WIN)],
                                  vbuf[b], semi[b]).wait()

        def fire_gather(b):
            pltpu.async_copy(proj.at[ebuf[b].at[pl.ds(EDGE_WIN, EDGE_WIN)]],
                             rows[b], semg[b])

        def wait_gather(b):
            pltpu.make_async_copy(proj.at[ebuf[b].at[pl.ds(EDGE_WIN, EDGE_WIN)]],
                                  rows[b], semg[b]).wait()

        def fire_scat(b):
            pltpu.async_copy(rows[b], acc.at[sidx[b]], sems[b], add=True)

        def wait_scat(b):
            pltpu.make_async_copy(rows[b], acc.at[sidx[b]], sems[b]).wait()

        def scale_and_stage(b):
            def grp(g, _):
                vv = vbuf[b][pl.ds(g * LANES, LANES)]
                sidx[b][pl.ds(g * LANES, LANES)] = ebuf[b][pl.ds(g * LANES, LANES)]
                for t in range(LANES):
                    k = g * LANES + t
                    v = jnp.full((LANES,), vv[t], jnp.float32)
                    for j in range(d // LANES):
                        rows[b][k, pl.ds(j * LANES, LANES)] = (
                            rows[b][k, pl.ds(j * LANES, LANES)] * v)
                return 0
            lax.fori_loop(0, n_grp, grp, 0)

        def zero_step(t, _):
            pltpu.sync_copy(zbuf, acc.at[pl.ds(s * rows_per_sub + t * zrows, zrows)])
            return 0
        lax.fori_loop(0, rows_per_sub // zrows, zero_step, 0)
        plsc.subcore_barrier()

        def body(w, b, peeled):
            b2 = (b + 2) % NBUF
            if peeled:
                wait_idx(w + 2, b2)
                if w >= 2:
                    wait_scat(b2)
                fire_gather(b2)
            else:
                @pl.when(w + 2 < n_win)
                def _():
                    wait_idx(w + 2, b2)
                    wait_scat(b2)
                    fire_gather(b2)
            wait_gather(b)
            scale_and_stage(b)
            fire_scat(b)
            if peeled:
                if w + NBUF < n_win:
                    fire_idx(w + NBUF, b)
            else:
                @pl.when(w + NBUF < n_win)
                def _():
                    fire_idx(w + NBUF, b)

        for b in range(NBUF):
            fire_idx(b, b)
        wait_idx(0, 0)
        fire_gather(0)
        wait_idx(1, 1)
        fire_gather(1)
        for w in range(n_peel):
            body(w, w % NBUF, peeled=True)

        def quad(t, _):
            for q in range(NBUF):
                w = n_peel + NBUF * t + q
                body(w, (n_peel + q) % NBUF, peeled=False)
            return 0
        lax.fori_loop(0, (n_win - n_peel) // NBUF, quad, 0)

        for wlast in range(n_win - NBUF, n_win):
            wait_scat(wlast % NBUF)
        plsc.subcore_barrier()

        pltpu.sync_copy(acc.at[pl.ds(s * rows_per_sub, rows_per_sub)],
                        part.at[c, pl.ds(s * rows_per_sub, rows_per_sub)])
        plsc.subcore_barrier()

    run_path(proj0, edges0, val0, part0)
    run_path(proj1, edges1, val1, part1)


def _spmm_both(proj0, proj1, edges0, val0, edges1, val1):
    n, d = proj0.shape
    e = edges0.shape[0] // 2
    n_pad = ((n + 8 * NS - 1) // (8 * NS)) * (8 * NS)
    e_per_w = e // (NC * NS)
    n_win = e_per_w // EDGE_WIN
    n_peel = NBUF + 1
    assert e_per_w % EDGE_WIN == 0
    assert n_win > n_peel and (n_win - n_peel) % NBUF == 0
    mesh = plsc.VectorSubcoreMesh(core_axis_name="c", subcore_axis_name="s",
                                  num_cores=NC, num_subcores=NS)
    kern = pl.kernel(
        functools.partial(_spmm_body, n_pad=n_pad, d=d, e=e),
        out_type=[
            jax.ShapeDtypeStruct((NC, n_pad, d), jnp.float32),
            jax.ShapeDtypeStruct((NC, n_pad, d), jnp.float32),
        ],
        mesh=mesh,
        scratch_types=(
            [pltpu.VMEM((2 * EDGE_WIN,), jnp.int32)] * NBUF
            + [pltpu.VMEM((EDGE_WIN,), jnp.float32)] * NBUF
            + [pltpu.VMEM((EDGE_WIN,), jnp.int32)] * NBUF
            + [pltpu.VMEM((EDGE_WIN, d), jnp.float32)] * NBUF
            + [pltpu.VMEM((8, d), jnp.float32)]
            + [pltpu.VMEM_SHARED((n_pad, d), jnp.float32)]
            + [pltpu.SemaphoreType.DMA] * (3 * NBUF)
        ),
    )
    return kern(proj0, proj1, edges0, val0, edges1, val1)


def _combine_body(masked_ref, part_ref, meta_ref, out_ref):
    meta = meta_ref[0, :]
    out_ref[...] = masked_ref[...] + STRENGTH * (
        part_ref[0] + part_ref[1] + meta[None, :])


def _combine(masked, part, meta_emb, block_rows=1000):
    n, d = masked.shape
    nb = n // block_rows
    return pl.pallas_call(
        _combine_body,
        grid=(nb,),
        in_specs=[
            pl.BlockSpec((block_rows, d), lambda b: (b, 0)),
            pl.BlockSpec((NC, block_rows, d), lambda b: (0, b, 0)),
            pl.BlockSpec((1, d), lambda b: (0, 0)),
        ],
        out_specs=pl.BlockSpec((block_rows, d), lambda b: (b, 0)),
        out_shape=jax.ShapeDtypeStruct((n, d), jnp.float32),
    )(masked, part, meta_emb)


def kernel(feat0, feat1, mp0_row, mp0_col, mp0_val, mp1_row, mp1_col, mp1_val,
           mask_idx0, mask_idx1, mask_token0, mask_token1,
           meta_emb0, meta_emb1, W0, W1):
    masked0, proj0, masked1, proj1 = _mask_project2(
        feat0, mask_idx0, mask_token0, W0, feat1, mask_idx1, mask_token1, W1)
    edges0 = _pack_edges(mp0_row, mp0_col)
    edges1 = _pack_edges(mp1_row, mp1_col)
    part0, part1 = _spmm_both(proj0, proj1, edges0, mp0_val, edges1, mp1_val)
    out0 = _combine(masked0, part0, meta_emb0)
    out1 = _combine(masked1, part1, meta_emb1)
    return (out0, out1)


# speedup vs baseline: 1.1400x; 1.1400x over previous
import functools

import jax
import jax.numpy as jnp
from jax import lax
from jax.experimental import pallas as pl
from jax.experimental.pallas import tpu as pltpu
from jax.experimental.pallas import tpu_sc as plsc

STRENGTH = 0.1

NC = 2
NS = 16
LANES = 16

NBUF = 4

EDGE_WIN = 80


def _mask_project_body(f0_ref, m0_ref, t0_ref, w0_ref, f1_ref, m1_ref, t1_ref,
                       w1_ref, mk0_ref, pj0_ref, mk1_ref, pj1_ref, *, block_rows):
    b = pl.program_id(0)

    def one(feat_ref, midx_ref, tok_ref, w_ref, masked_ref, proj_ref):
        feat = feat_ref[...]
        midx = midx_ref[0, :]
        rows = b * block_rows + lax.broadcasted_iota(
            jnp.int32, (block_rows, midx.shape[0]), 0)
        is_masked = jnp.any(rows == midx[None, :], axis=1)
        tok = tok_ref[0, :]
        masked = jnp.where(is_masked[:, None], tok[None, :], feat)
        masked_ref[...] = masked
        proj_ref[...] = lax.dot_general(
            masked, w_ref[...], (((1,), (1,)), ((), ())),
            preferred_element_type=jnp.float32,
            precision=lax.Precision.HIGHEST)

    one(f0_ref, m0_ref, t0_ref, w0_ref, mk0_ref, pj0_ref)
    one(f1_ref, m1_ref, t1_ref, w1_ref, mk1_ref, pj1_ref)


def _mask_project2(feat0, midx0, tok0, w0, feat1, midx1, tok1, w1,
                   block_rows=1000):
    n, d = feat0.shape
    nb = n // block_rows
    nmask = midx0.shape[0]
    fspec = pl.BlockSpec((block_rows, d), lambda b: (b, 0))
    mspec = pl.BlockSpec((1, nmask), lambda b: (0, 0))
    tspec = pl.BlockSpec((1, d), lambda b: (0, 0))
    wspec = pl.BlockSpec((d, d), lambda b: (0, 0))
    return pl.pallas_call(
        functools.partial(_mask_project_body, block_rows=block_rows),
        grid=(nb,),
        in_specs=[fspec, mspec, tspec, wspec, fspec, mspec, tspec, wspec],
        out_specs=[fspec, fspec, fspec, fspec],
        out_shape=[jax.ShapeDtypeStruct((n, d), jnp.float32)] * 4,
    )(feat0, midx0.reshape(1, nmask), tok0, w0,
      feat1, midx1.reshape(1, nmask), tok1, w1)


def _spmm_body(proj0, proj1, row0, col0, val0, row1, col1, val1,
               part0, part1,
               *refs, n_pad, d, e):
    ridx = refs[0:NBUF]
    cidx = refs[NBUF:2 * NBUF]
    vbuf = refs[2 * NBUF:3 * NBUF]
    sidx = refs[3 * NBUF:4 * NBUF]
    rows = refs[4 * NBUF:5 * NBUF]
    zbuf = refs[5 * NBUF]
    acc = refs[5 * NBUF + 1]
    semi = refs[5 * NBUF + 2:5 * NBUF + 2 + NBUF]
    semg = refs[5 * NBUF + 2 + NBUF:5 * NBUF + 2 + 2 * NBUF]
    sems = refs[5 * NBUF + 2 + 2 * NBUF:5 * NBUF + 2 + 3 * NBUF]

    c = lax.axis_index("c")
    s = lax.axis_index("s")
    wid = c * NS + s
    e_per_w = e // (NC * NS)
    n_win = e_per_w // EDGE_WIN
    rows_per_sub = n_pad // NS
    zrows = zbuf.shape[0]
    n_grp = EDGE_WIN // LANES
    n_peel = NBUF + 1

    def zfill(r, _):
        for j in range(d // LANES):
            zbuf[r, pl.ds(j * LANES, LANES)] = jnp.zeros((LANES,), jnp.float32)
        return 0
    lax.fori_loop(0, zrows, zfill, 0)

    def run_path(proj, row, col, val, part):
        base0 = wid * e_per_w

        def fire_idx(w, b):
            off = base0 + w * EDGE_WIN
            pltpu.async_copy(row.at[pl.ds(off, EDGE_WIN)], ridx[b], semi[b])
            pltpu.async_copy(col.at[pl.ds(off, EDGE_WIN)], cidx[b], semi[b])
            pltpu.async_copy(val.at[pl.ds(off, EDGE_WIN)], vbuf[b], semi[b])

        def wait_idx(w, b):
            off = base0 + w * EDGE_WIN
            pltpu.make_async_copy(row.at[pl.ds(off, EDGE_WIN)], ridx[b], semi[b]).wait()
            pltpu.make_async_copy(col.at[pl.ds(off, EDGE_WIN)], cidx[b], semi[b]).wait()
            pltpu.make_async_copy(val.at[pl.ds(off, EDGE_WIN)], vbuf[b], semi[b]).wait()

        def fire_gather(b):
            pltpu.async_copy(proj.at[cidx[b]], rows[b], semg[b])

        def wait_gather(b):
            pltpu.make_async_copy(proj.at[cidx[b]], rows[b], semg[b]).wait()

        def fire_scat(b):
            pltpu.async_copy(rows[b], acc.at[sidx[b]], sems[b], add=True)

        def wait_scat(b):
            pltpu.make_async_copy(rows[b], acc.at[sidx[b]], sems[b]).wait()

        def scale_and_stage(b):
            def grp(g, _):
                vv = vbuf[b][pl.ds(g * LANES, LANES)]
                sidx[b][pl.ds(g * LANES, LANES)] = ridx[b][pl.ds(g * LANES, LANES)]
                for t in range(LANES):
                    k = g * LANES + t
                    v = jnp.full((LANES,), vv[t], jnp.float32)
                    for j in range(d // LANES):
                        rows[b][k, pl.ds(j * LANES, LANES)] = (
                            rows[b][k, pl.ds(j * LANES, LANES)] * v)
                return 0
            lax.fori_loop(0, n_grp, grp, 0)

        def zero_fire(t, _):
            pltpu.async_copy(zbuf, acc.at[pl.ds(s * rows_per_sub + t * zrows,
                                                zrows)], semi[0])
            return 0
        lax.fori_loop(0, rows_per_sub // zrows, zero_fire, 0)

        def zero_drain(t, _):
            pltpu.make_async_copy(zbuf, acc.at[pl.ds(s * rows_per_sub + t * zrows,
                                                     zrows)], semi[0]).wait()
            return 0
        lax.fori_loop(0, rows_per_sub // zrows, zero_drain, 0)
        plsc.subcore_barrier()

        def body(w, b, peeled):
            b2 = (b + 2) % NBUF
            if peeled:
                if w + 2 < n_win:
                    wait_idx(w + 2, b2)
                    if w >= 2:
                        wait_scat(b2)
                    fire_gather(b2)
            else:
                @pl.when(w + 2 < n_win)
                def _():
                    wait_idx(w + 2, b2)
                    wait_scat(b2)
                    fire_gather(b2)
            wait_gather(b)
            scale_and_stage(b)
            fire_scat(b)
            if peeled:
                if w + NBUF < n_win:
                    fire_idx(w + NBUF, b)
            else:
                @pl.when(w + NBUF < n_win)
                def _():
                    fire_idx(w + NBUF, b)

        for b in range(NBUF):
            fire_idx(b, b)
        wait_idx(0, 0)
        fire_gather(0)
        wait_idx(1, 1)
        fire_gather(1)
        for w in range(n_peel):
            body(w, w % NBUF, peeled=True)

        def quad(t, _):
            for q in range(NBUF):
                w = n_peel + NBUF * t + q
                body(w, (n_peel + q) % NBUF, peeled=False)
            return 0
        lax.fori_loop(0, (n_win - n_peel) // NBUF, quad, 0)

        for wlast in range(n_win - NBUF, n_win):
            wait_scat(wlast % NBUF)
        plsc.subcore_barrier()

        pltpu.sync_copy(acc.at[pl.ds(s * rows_per_sub, rows_per_sub)],
                        part.at[c, pl.ds(s * rows_per_sub, rows_per_sub)])
        plsc.subcore_barrier()

    run_path(proj0, row0, col0, val0, part0)
    run_path(proj1, row1, col1, val1, part1)


def _spmm_both(proj0, proj1, row0, col0, val0, row1, col1, val1):
    n, d = proj0.shape
    e = row0.shape[0]
    n_pad = ((n + 8 * NS - 1) // (8 * NS)) * (8 * NS)
    e_per_w = e // (NC * NS)
    n_win = e_per_w // EDGE_WIN
    n_peel = NBUF + 1
    assert e_per_w % EDGE_WIN == 0
    assert n_win > n_peel and (n_win - n_peel) % NBUF == 0
    mesh = plsc.VectorSubcoreMesh(core_axis_name="c", subcore_axis_name="s",
                                  num_cores=NC, num_subcores=NS)
    kern = pl.kernel(
        functools.partial(_spmm_body, n_pad=n_pad, d=d, e=e),
        out_type=[
            jax.ShapeDtypeStruct((NC, n_pad, d), jnp.float32),
            jax.ShapeDtypeStruct((NC, n_pad, d), jnp.float32),
        ],
        mesh=mesh,
        scratch_types=(
            [pltpu.VMEM((EDGE_WIN,), jnp.int32)] * NBUF
            + [pltpu.VMEM((EDGE_WIN,), jnp.int32)] * NBUF
            + [pltpu.VMEM((EDGE_WIN,), jnp.float32)] * NBUF
            + [pltpu.VMEM((EDGE_WIN,), jnp.int32)] * NBUF
            + [pltpu.VMEM((EDGE_WIN, d), jnp.float32)] * NBUF
            + [pltpu.VMEM((8, d), jnp.float32)]
            + [pltpu.VMEM_SHARED((n_pad, d), jnp.float32)]
            + [pltpu.SemaphoreType.DMA] * (3 * NBUF)
        ),
    )
    return kern(proj0, proj1, row0, col0, val0, row1, col1, val1)


def _combine_body(masked_ref, part_ref, meta_ref, out_ref):
    meta = meta_ref[0, :]
    out_ref[...] = masked_ref[...] + STRENGTH * (
        part_ref[0] + part_ref[1] + meta[None, :])


def _combine(masked, part, meta_emb, block_rows=1000):
    n, d = masked.shape
    nb = n // block_rows
    return pl.pallas_call(
        _combine_body,
        grid=(nb,),
        in_specs=[
            pl.BlockSpec((block_rows, d), lambda b: (b, 0)),
            pl.BlockSpec((NC, block_rows, d), lambda b: (0, b, 0)),
            pl.BlockSpec((1, d), lambda b: (0, 0)),
        ],
        out_specs=pl.BlockSpec((block_rows, d), lambda b: (b, 0)),
        out_shape=jax.ShapeDtypeStruct((n, d), jnp.float32),
    )(masked, part, meta_emb)


def kernel(feat0, feat1, mp0_row, mp0_col, mp0_val, mp1_row, mp1_col, mp1_val,
           mask_idx0, mask_idx1, mask_token0, mask_token1,
           meta_emb0, meta_emb1, W0, W1):
    masked0, proj0, masked1, proj1 = _mask_project2(
        feat0, mask_idx0, mask_token0, W0, feat1, mask_idx1, mask_token1, W1)
    part0, part1 = _spmm_both(proj0, proj1, mp0_row, mp0_col, mp0_val,
                              mp1_row, mp1_col, mp1_val)
    out0 = _combine(masked0, part0, meta_emb0)
    out1 = _combine(masked1, part1, meta_emb1)
    return (out0, out1)


# speedup vs baseline: 1.1670x; 1.0237x over previous
import functools

import jax
import jax.numpy as jnp
from jax import lax
from jax.experimental import pallas as pl
from jax.experimental.pallas import tpu as pltpu
from jax.experimental.pallas import tpu_sc as plsc

STRENGTH = 0.1

NC = 2
NS = 16
LANES = 16

NBUF = 4

EDGE_WIN = 80


def _mask_project_body(f0_ref, m0_ref, t0_ref, w0_ref, f1_ref, m1_ref, t1_ref,
                       w1_ref, mk0_ref, pj0_ref, mk1_ref, pj1_ref, *, block_rows):
    b = pl.program_id(0)

    def one(feat_ref, midx_ref, tok_ref, w_ref, masked_ref, proj_ref):
        feat = feat_ref[...]
        midx = midx_ref[0, :]
        rows = b * block_rows + lax.broadcasted_iota(
            jnp.int32, (block_rows, midx.shape[0]), 0)
        is_masked = jnp.any(rows == midx[None, :], axis=1)
        tok = tok_ref[0, :]
        masked = jnp.where(is_masked[:, None], tok[None, :], feat)
        masked_ref[...] = masked
        proj_ref[...] = lax.dot_general(
            masked, w_ref[...], (((1,), (1,)), ((), ())),
            preferred_element_type=jnp.float32,
            precision=lax.Precision.HIGHEST)

    one(f0_ref, m0_ref, t0_ref, w0_ref, mk0_ref, pj0_ref)
    one(f1_ref, m1_ref, t1_ref, w1_ref, mk1_ref, pj1_ref)


def _mask_project2(feat0, midx0, tok0, w0, feat1, midx1, tok1, w1,
                   block_rows=1000):
    n, d = feat0.shape
    nb = n // block_rows
    nmask = midx0.shape[0]
    fspec = pl.BlockSpec((block_rows, d), lambda b: (b, 0))
    mspec = pl.BlockSpec((1, nmask), lambda b: (0, 0))
    tspec = pl.BlockSpec((1, d), lambda b: (0, 0))
    wspec = pl.BlockSpec((d, d), lambda b: (0, 0))
    return pl.pallas_call(
        functools.partial(_mask_project_body, block_rows=block_rows),
        grid=(nb,),
        in_specs=[fspec, mspec, tspec, wspec, fspec, mspec, tspec, wspec],
        out_specs=[fspec, fspec, fspec, fspec],
        out_shape=[jax.ShapeDtypeStruct((n, d), jnp.float32)] * 4,
    )(feat0, midx0.reshape(1, nmask), tok0, w0,
      feat1, midx1.reshape(1, nmask), tok1, w1)


def _spmm_body(proj0, proj1, row0, col0, val0, row1, col1, val1,
               part0, part1,
               *refs, n_pad, d, e):
    ridx = refs[0:NBUF]
    cidx = refs[NBUF:2 * NBUF]
    vbuf = refs[2 * NBUF:3 * NBUF]
    sidx = refs[3 * NBUF:4 * NBUF]
    rows = refs[4 * NBUF:5 * NBUF]
    zbuf = refs[5 * NBUF]
    acc = refs[5 * NBUF + 1]
    semi = refs[5 * NBUF + 2:5 * NBUF + 2 + NBUF]
    semg = refs[5 * NBUF + 2 + NBUF:5 * NBUF + 2 + 2 * NBUF]
    sems = refs[5 * NBUF + 2 + 2 * NBUF:5 * NBUF + 2 + 3 * NBUF]

    c = lax.axis_index("c")
    s = lax.axis_index("s")
    wid = c * NS + s
    e_per_w = e // (NC * NS)
    n_win = e_per_w // EDGE_WIN
    rows_per_sub = n_pad // NS
    zrows = zbuf.shape[0]
    n_grp = EDGE_WIN // LANES
    n_peel = NBUF + 1

    def zfill(r, _):
        for j in range(d // LANES):
            zbuf[r, pl.ds(j * LANES, LANES)] = jnp.zeros((LANES,), jnp.float32)
        return 0
    lax.fori_loop(0, zrows, zfill, 0)

    def run_path(proj, row, col, val, part):
        base0 = wid * e_per_w

        def fire_idx(w, b):
            off = base0 + w * EDGE_WIN
            pltpu.async_copy(row.at[pl.ds(off, EDGE_WIN)], ridx[b], semi[b])
            pltpu.async_copy(col.at[pl.ds(off, EDGE_WIN)], cidx[b], semi[b])
            pltpu.async_copy(val.at[pl.ds(off, EDGE_WIN)], vbuf[b], semi[b])

        def wait_idx(w, b):
            off = base0 + w * EDGE_WIN
            pltpu.make_async_copy(row.at[pl.ds(off, EDGE_WIN)], ridx[b], semi[b]).wait()
            pltpu.make_async_copy(col.at[pl.ds(off, EDGE_WIN)], cidx[b], semi[b]).wait()
            pltpu.make_async_copy(val.at[pl.ds(off, EDGE_WIN)], vbuf[b], semi[b]).wait()

        def fire_gather(b):
            pltpu.async_copy(proj.at[cidx[b]], rows[b], semg[b])

        def wait_gather(b):
            pltpu.make_async_copy(proj.at[cidx[b]], rows[b], semg[b]).wait()

        def fire_scat(b):
            pltpu.async_copy(rows[b], acc.at[sidx[b]], sems[b], add=True)

        def wait_scat(b):
            pltpu.make_async_copy(rows[b], acc.at[sidx[b]], sems[b]).wait()

        def scale_and_stage(b):
            def grp(g, _):
                vv = vbuf[b][pl.ds(g * LANES, LANES)]
                sidx[b][pl.ds(g * LANES, LANES)] = ridx[b][pl.ds(g * LANES, LANES)]
                for t in range(LANES):
                    k = g * LANES + t
                    v = jnp.full((LANES,), vv[t], jnp.float32)
                    for j in range(d // LANES):
                        rows[b][k, pl.ds(j * LANES, LANES)] = (
                            rows[b][k, pl.ds(j * LANES, LANES)] * v)
                return 0
            lax.fori_loop(0, n_grp, grp, 0)

        def zero_fire(t, _):
            pltpu.async_copy(zbuf, acc.at[pl.ds(s * rows_per_sub + t * zrows,
                                                zrows)], semi[0])
            return 0
        lax.fori_loop(0, rows_per_sub // zrows, zero_fire, 0)

        def zero_drain(t, _):
            pltpu.make_async_copy(zbuf, acc.at[pl.ds(s * rows_per_sub + t * zrows,
                                                     zrows)], semi[0]).wait()
            return 0
        lax.fori_loop(0, rows_per_sub // zrows, zero_drain, 0)
        plsc.subcore_barrier()

        def body(w, b, peeled):
            b2 = (b + 2) % NBUF
            if peeled:
                if w + 2 < n_win:
                    wait_idx(w + 2, b2)
                    if w >= 2:
                        wait_scat(b2)
                    fire_gather(b2)
            else:
                @pl.when(w + 2 < n_win)
                def _():
                    wait_idx(w + 2, b2)
                    wait_scat(b2)
                    fire_gather(b2)
            wait_gather(b)
            scale_and_stage(b)
            fire_scat(b)
            if peeled:
                if w + NBUF < n_win:
                    fire_idx(w + NBUF, b)
            else:
                @pl.when(w + NBUF < n_win)
                def _():
                    fire_idx(w + NBUF, b)

        for b in range(NBUF):
            fire_idx(b, b)
        wait_idx(0, 0)
        fire_gather(0)
        wait_idx(1, 1)
        fire_gather(1)
        for w in range(n_peel):
            body(w, w % NBUF, peeled=True)

        def quad(t, _):
            for q in range(NBUF):
                w = n_peel + NBUF * t + q
                body(w, (n_peel + q) % NBUF, peeled=False)
            return 0
        lax.fori_loop(0, (n_win - n_peel) // NBUF, quad, 0)

        for wlast in range(n_win - NBUF, n_win):
            wait_scat(wlast % NBUF)
        plsc.subcore_barrier()

        pltpu.sync_copy(acc.at[pl.ds(s * rows_per_sub, rows_per_sub)],
                        part.at[c, pl.ds(s * rows_per_sub, rows_per_sub)])
        plsc.subcore_barrier()

    run_path(proj0, row0, col0, val0, part0)
    run_path(proj1, row1, col1, val1, part1)


def _spmm_both(proj0, proj1, row0, col0, val0, row1, col1, val1):
    n, d = proj0.shape
    e = row0.shape[0]
    n_pad = ((n + 8 * NS - 1) // (8 * NS)) * (8 * NS)
    e_per_w = e // (NC * NS)
    n_win = e_per_w // EDGE_WIN
    n_peel = NBUF + 1
    assert e_per_w % EDGE_WIN == 0
    assert n_win > n_peel and (n_win - n_peel) % NBUF == 0
    mesh = plsc.VectorSubcoreMesh(core_axis_name="c", subcore_axis_name="s",
                                  num_cores=NC, num_subcores=NS)
    kern = pl.kernel(
        functools.partial(_spmm_body, n_pad=n_pad, d=d, e=e),
        out_type=[
            jax.ShapeDtypeStruct((NC, n_pad, d), jnp.float32),
            jax.ShapeDtypeStruct((NC, n_pad, d), jnp.float32),
        ],
        mesh=mesh,
        scratch_types=(
            [pltpu.VMEM((EDGE_WIN,), jnp.int32)] * NBUF
            + [pltpu.VMEM((EDGE_WIN,), jnp.int32)] * NBUF
            + [pltpu.VMEM((EDGE_WIN,), jnp.float32)] * NBUF
            + [pltpu.VMEM((EDGE_WIN,), jnp.int32)] * NBUF
            + [pltpu.VMEM((EDGE_WIN, d), jnp.float32)] * NBUF
            + [pltpu.VMEM((8, d), jnp.float32)]
            + [pltpu.VMEM_SHARED((n_pad, d), jnp.float32)]
            + [pltpu.SemaphoreType.DMA] * (3 * NBUF)
        ),
    )
    return kern(proj0, proj1, row0, col0, val0, row1, col1, val1)


def _combine_body(mk0_ref, pt0_ref, mt0_ref, mk1_ref, pt1_ref, mt1_ref,
                  out0_ref, out1_ref):
    def one(masked_ref, part_ref, meta_ref, out_ref):
        meta = meta_ref[0, :]
        out_ref[...] = masked_ref[...] + STRENGTH * (
            part_ref[0] + part_ref[1] + meta[None, :])

    one(mk0_ref, pt0_ref, mt0_ref, out0_ref)
    one(mk1_ref, pt1_ref, mt1_ref, out1_ref)


def _combine2(masked0, part0, meta0, masked1, part1, meta1, block_rows=1000):
    n, d = masked0.shape
    nb = n // block_rows
    mspec = pl.BlockSpec((block_rows, d), lambda b: (b, 0))
    pspec = pl.BlockSpec((NC, block_rows, d), lambda b: (0, b, 0))
    espec = pl.BlockSpec((1, d), lambda b: (0, 0))
    return pl.pallas_call(
        _combine_body,
        grid=(nb,),
        in_specs=[mspec, pspec, espec, mspec, pspec, espec],
        out_specs=[mspec, mspec],
        out_shape=[jax.ShapeDtypeStruct((n, d), jnp.float32)] * 2,
    )(masked0, part0, meta0, masked1, part1, meta1)


def kernel(feat0, feat1, mp0_row, mp0_col, mp0_val, mp1_row, mp1_col, mp1_val,
           mask_idx0, mask_idx1, mask_token0, mask_token1,
           meta_emb0, meta_emb1, W0, W1):
    masked0, proj0, masked1, proj1 = _mask_project2(
        feat0, mask_idx0, mask_token0, W0, feat1, mask_idx1, mask_token1, W1)
    part0, part1 = _spmm_both(proj0, proj1, mp0_row, mp0_col, mp0_val,
                              mp1_row, mp1_col, mp1_val)
    out0, out1 = _combine2(masked0, part0, meta_emb0, masked1, part1, meta_emb1)
    return (out0, out1)
